# baseline (device time: 12425 ns/iter reference)
import jax
import jax.numpy as jnp
from jax import lax
from jax.experimental import pallas as pl
from jax.experimental.pallas import tpu as pltpu

N_DEV = 4
BLK = 256
ORDER = (1, 3, 2)


def kernel(x, w_mat):
    k_total, k_per = x.shape
    _, n = w_mat.shape
    m_per = k_total // N_DEV

    def body(x_ref, w_hbm, out_ref, xb_ref, w_ref, wb_ref, comm_ref,
             send_sems, recv_sems, ready_sems, w_sem):
        my = lax.axis_index("i")

        barrier_sem = pltpu.get_barrier_semaphore()

        for d in (1, 3):
            pl.semaphore_signal(
                ready_sems.at[3 - d], inc=1,
                device_id=((my + d) % N_DEV,),
                device_id_type=pl.DeviceIdType.MESH,
            )
        pl.semaphore_signal(
            barrier_sem, inc=1,
            device_id=((my + 2) % N_DEV,),
            device_id_type=pl.DeviceIdType.MESH,
        )

        wcopy = pltpu.make_async_copy(w_hbm, w_ref, w_sem)
        wcopy.start()

        xb_ref[...] = x_ref[...].astype(jnp.bfloat16)

        rdmas = {}
        for d in ORDER:
            j = (my + d) % N_DEV
            if d == 2:
                pl.semaphore_wait(barrier_sem, 1)
            else:
                pl.semaphore_wait(ready_sems.at[d - 1], 1)
            rdma = pltpu.make_async_remote_copy(
                src_ref=xb_ref.at[pl.ds(j * m_per, m_per), :],
                dst_ref=comm_ref.at[d - 1],
                send_sem=send_sems.at[d - 1],
                recv_sem=recv_sems.at[d - 1],
                device_id=(j,),
                device_id_type=pl.DeviceIdType.MESH,
            )
            rdma.start()
            rdmas[d] = rdma

        wcopy.wait()
        wb_ref[...] = w_ref[...].astype(jnp.bfloat16)
        acc = jnp.dot(
            xb_ref[pl.ds(my * m_per, m_per), :],
            wb_ref[pl.ds(my * BLK, BLK), :],
            preferred_element_type=jnp.float32,
        )

        for d in ORDER:
            rdmas[d].wait_recv()
            s = (my - d) % N_DEV
            acc = acc + jnp.dot(
                comm_ref[d - 1], wb_ref[pl.ds(s * BLK, BLK), :],
                preferred_element_type=jnp.float32,
            )
        out_ref[...] = jnp.maximum(acc, 0.0).astype(jnp.bfloat16)

        for d in ORDER:
            rdmas[d].wait_send()

    return pl.pallas_call(
        body,
        out_shape=jax.ShapeDtypeStruct((m_per, n), jnp.bfloat16),
        in_specs=[
            pl.BlockSpec(memory_space=pltpu.VMEM),
            pl.BlockSpec(memory_space=pl.ANY),
        ],
        out_specs=pl.BlockSpec(memory_space=pltpu.VMEM),
        scratch_shapes=[
            pltpu.VMEM((k_total, k_per), jnp.bfloat16),
            pltpu.VMEM((k_total, n), jnp.float32),
            pltpu.VMEM((k_total, n), jnp.bfloat16),
            pltpu.VMEM((N_DEV - 1, m_per, k_per), jnp.bfloat16),
            pltpu.SemaphoreType.DMA((N_DEV - 1,)),
            pltpu.SemaphoreType.DMA((N_DEV - 1,)),
            pltpu.SemaphoreType.REGULAR((N_DEV - 1,)),
            pltpu.SemaphoreType.DMA,
        ],
        compiler_params=pltpu.CompilerParams(collective_id=0),
    )(x, w_mat)


# device time: 12382 ns/iter; 1.0035x vs baseline; 1.0035x over previous
import jax
import jax.numpy as jnp
from jax import lax
from jax.experimental import pallas as pl
from jax.experimental.pallas import tpu as pltpu

N_DEV = 4
BLK = 256
HALF = 128


def kernel(x, w_mat):
    k_total, k_per = x.shape
    _, n = w_mat.shape
    m_per = k_total // N_DEV

    def body(x_ref, w_ref, out_ref, xb_ref, wb_ref, comm_ref,
             send_sems, recv_sems, ready_sems, dsend_sems, drecv_sems):
        my = lax.axis_index("i")

        barrier_sem = pltpu.get_barrier_semaphore()
        for d in (1, 3):
            pl.semaphore_signal(
                ready_sems.at[3 - d], inc=1,
                device_id=((my + d) % N_DEV,),
                device_id_type=pl.DeviceIdType.MESH,
            )
        pl.semaphore_signal(
            barrier_sem, inc=1,
            device_id=((my + 2) % N_DEV,),
            device_id_type=pl.DeviceIdType.MESH,
        )

        rdmas = {}
        for d in (1, 3):
            j = (my + d) % N_DEV
            sl = pl.ds(j * m_per, m_per)
            xb_ref[sl, :] = x_ref[sl, :].astype(jnp.bfloat16)
            pl.semaphore_wait(ready_sems.at[d - 1], 1)
            rdma = pltpu.make_async_remote_copy(
                src_ref=xb_ref.at[pl.ds(j * m_per, m_per), :],
                dst_ref=comm_ref.at[d - 1],
                send_sem=send_sems.at[d - 1],
                recv_sem=recv_sems.at[d - 1],
                device_id=(j,),
                device_id_type=pl.DeviceIdType.MESH,
            )
            rdma.start()
            rdmas[d] = rdma

        jd = (my + 2) % N_DEV
        sl = pl.ds(jd * m_per, m_per)
        xb_ref[sl, :] = x_ref[sl, :].astype(jnp.bfloat16)
        pl.semaphore_wait(barrier_sem, 1)
        drdmas = []
        for h in range(2):
            rdma = pltpu.make_async_remote_copy(
                src_ref=xb_ref.at[pl.ds(jd * m_per + h * HALF, HALF), :],
                dst_ref=comm_ref.at[1, pl.ds(h * HALF, HALF), :],
                send_sem=dsend_sems.at[h],
                recv_sem=drecv_sems.at[h],
                device_id=(jd,),
                device_id_type=pl.DeviceIdType.MESH,
            )
            rdma.start()
            drdmas.append(rdma)

        sl = pl.ds(my * m_per, m_per)
        xb_ref[sl, :] = x_ref[sl, :].astype(jnp.bfloat16)
        wb_ref[...] = w_ref[...].astype(jnp.bfloat16)
        acc = jnp.dot(
            xb_ref[pl.ds(my * m_per, m_per), :],
            wb_ref[pl.ds(my * BLK, BLK), :],
            preferred_element_type=jnp.float32,
        )

        for d in (1, 3):
            rdmas[d].wait_recv()
            s = (my - d) % N_DEV
            acc = acc + jnp.dot(
                comm_ref[d - 1], wb_ref[pl.ds(s * BLK, BLK), :],
                preferred_element_type=jnp.float32,
            )

        sd = (my - 2) % N_DEV
        for h in range(2):
            drdmas[h].wait_recv()
            part = acc[h * HALF:(h + 1) * HALF, :] + jnp.dot(
                comm_ref[1, h * HALF:(h + 1) * HALF, :],
                wb_ref[pl.ds(sd * BLK, BLK), :],
                preferred_element_type=jnp.float32,
            )
            out_ref[h * HALF:(h + 1) * HALF, :] = jnp.maximum(
                part, 0.0
            ).astype(jnp.bfloat16)

        for d in (1, 3):
            rdmas[d].wait_send()
        for h in range(2):
            drdmas[h].wait_send()

    return pl.pallas_call(
        body,
        out_shape=jax.ShapeDtypeStruct((m_per, n), jnp.bfloat16),
        in_specs=[
            pl.BlockSpec(memory_space=pltpu.VMEM),
            pl.BlockSpec(memory_space=pltpu.VMEM),
        ],
        out_specs=pl.BlockSpec(memory_space=pltpu.VMEM),
        scratch_shapes=[
            pltpu.VMEM((k_total, k_per), jnp.bfloat16),
            pltpu.VMEM((k_total, n), jnp.bfloat16),
            pltpu.VMEM((N_DEV - 1, m_per, k_per), jnp.bfloat16),
            pltpu.SemaphoreType.DMA((N_DEV - 1,)),
            pltpu.SemaphoreType.DMA((N_DEV - 1,)),
            pltpu.SemaphoreType.REGULAR((N_DEV - 1,)),
            pltpu.SemaphoreType.DMA((2,)),
            pltpu.SemaphoreType.DMA((2,)),
        ],
        compiler_params=pltpu.CompilerParams(collective_id=0),
    )(x, w_mat)
